# Initial kernel scaffold; baseline (speedup 1.0000x reference)
import functools

import jax
import jax.numpy as jnp
from jax import lax
from jax.experimental import pallas as pl
from jax.experimental.pallas import tpu as pltpu
from jax.experimental.pallas import tpu_sc as plsc

B = 4096
HID = 128
NNB = 20

_NC = 2
_NS = 16
_NW = _NC * _NS
_BW = B // _NW
_CH = 16
_NCHUNK = _BW // _CH
_LANES = 8


def _sc_gather_reduce(table, node_ids, neigh_flat):
    mesh = plsc.VectorSubcoreMesh(core_axis_name="c", subcore_axis_name="s")

    @functools.partial(
        pl.kernel,
        mesh=mesh,
        out_type=[
            jax.ShapeDtypeStruct((B, HID), jnp.float32),
            jax.ShapeDtypeStruct((B, HID), jnp.float32),
        ],
        scratch_types=[
            pltpu.VMEM((_BW,), jnp.int32),
            pltpu.VMEM((_BW * NNB,), jnp.int32),
            pltpu.VMEM((_BW, HID), jnp.float32),
            pltpu.VMEM((_CH * NNB, HID), jnp.float32),
            pltpu.VMEM((_CH * NNB, HID), jnp.float32),
            pltpu.VMEM((_BW, HID), jnp.float32),
            pltpu.SemaphoreType.DMA,
            pltpu.SemaphoreType.DMA,
            pltpu.SemaphoreType.DMA,
        ],
    )
    def k(table_hbm, ids_hbm, nbr_hbm, self_out, sum_out,
          idx_v, nidx_v, self_v, bufa, bufb, sum_v, sem_self, sema, semb):
        wid = lax.axis_index("s") * _NC + lax.axis_index("c")
        base = wid * _BW

        pltpu.sync_copy(ids_hbm.at[pl.ds(base, _BW)], idx_v)
        pltpu.sync_copy(nbr_hbm.at[pl.ds(base * NNB, _BW * NNB)], nidx_v)

        self_cp = pltpu.async_copy(table_hbm.at[idx_v], self_v, sem_self)

        bufs = (bufa, bufb)
        sems = (sema, semb)
        pend = [None, None]
        pend[0] = pltpu.async_copy(
            table_hbm.at[nidx_v.at[pl.ds(0, _CH * NNB)]], bufs[0], sems[0])

        for c in range(_NCHUNK):
            cur = c % 2
            if c + 1 < _NCHUNK:
                nxt = (c + 1) % 2
                pend[nxt] = pltpu.async_copy(
                    table_hbm.at[nidx_v.at[pl.ds((c + 1) * _CH * NNB, _CH * NNB)]],
                    bufs[nxt], sems[nxt])
            pend[cur].wait()
            buf = bufs[cur]

            def body(i, carry, buf=buf, c=c):
                accs = [jnp.zeros((16,), jnp.float32) for _ in range(_LANES)]
                for j in range(NNB):
                    row = i * NNB + j
                    for g in range(_LANES):
                        accs[g] = accs[g] + buf[row, pl.ds(g * 16, 16)]
                for g in range(_LANES):
                    sum_v[c * _CH + i, pl.ds(g * 16, 16)] = accs[g]
                return carry

            lax.fori_loop(0, _CH, body, 0)

        self_cp.wait()
        pltpu.sync_copy(self_v, self_out.at[pl.ds(base, _BW)])
        pltpu.sync_copy(sum_v, sum_out.at[pl.ds(base, _BW)])

    return k(table, node_ids, neigh_flat)


_TBLK = 512
_TGRID = B // _TBLK


def _tc_body(self_ref, nsum_ref, tb_ref,
             wsp1_ref, wsp2_ref, bsp_ref, wt_ref, bt_ref,
             wf1_ref, wf2_ref, bf_ref,
             w1a_ref, w1b_ref, w1c_ref, b1_ref, w2_ref, b2_ref, out_ref):
    spatial = jnp.maximum(
        jnp.dot(self_ref[...], wsp1_ref[...], preferred_element_type=jnp.float32)
        + jnp.dot(nsum_ref[...], wsp2_ref[...], preferred_element_type=jnp.float32)
        + bsp_ref[...], 0.0)
    temporal = jnp.cos(tb_ref[...] * wt_ref[...] + bt_ref[...])
    fused = jnp.maximum(
        jnp.dot(spatial, wf1_ref[...], preferred_element_type=jnp.float32)
        + jnp.dot(temporal, wf2_ref[...], preferred_element_type=jnp.float32)
        + bf_ref[...], 0.0)
    h = jnp.maximum(
        jnp.dot(spatial, w1a_ref[...], preferred_element_type=jnp.float32)
        + jnp.dot(temporal, w1b_ref[...], preferred_element_type=jnp.float32)
        + jnp.dot(fused, w1c_ref[...], preferred_element_type=jnp.float32)
        + b1_ref[...], 0.0)
    out_ref[...] = (
        jnp.dot(h, w2_ref[...], preferred_element_type=jnp.float32)
        + b2_ref[...])


def _tc_dense(self_emb, nsum, tb, wsp1, wsp2, bsp, wt, bt,
              wf1, wf2, bf, w1a, w1b, w1c, b1, w2, b2):
    row_spec = pl.BlockSpec((_TBLK, HID), lambda i: (i, 0))

    def wspec(shape):
        return pl.BlockSpec(shape, lambda i, _s=len(shape): (0,) * _s)

    return pl.pallas_call(
        _tc_body,
        grid=(_TGRID,),
        in_specs=[
            row_spec, row_spec, row_spec,
            wspec(wsp1.shape), wspec(wsp2.shape), wspec(bsp.shape),
            wspec(wt.shape), wspec(bt.shape),
            wspec(wf1.shape), wspec(wf2.shape), wspec(bf.shape),
            wspec(w1a.shape), wspec(w1b.shape), wspec(w1c.shape),
            wspec(b1.shape), wspec(w2.shape), wspec(b2.shape),
        ],
        out_specs=row_spec,
        out_shape=jax.ShapeDtypeStruct((B, HID), jnp.float32),
    )(self_emb, nsum, tb, wsp1, wsp2, bsp, wt, bt,
      wf1, wf2, bf, w1a, w1b, w1c, b1, w2, b2)


def kernel(node_ids, node_interact_times, raw_node_features, node_emb_table,
           neighbor_ids, W_sp, b_sp, w_t, b_t, W_f, b_f, W1, b1, W2, b2):
    ids = node_ids.astype(jnp.int32)
    neigh_flat = neighbor_ids.astype(jnp.int32).reshape(B * NNB)

    self_emb, nsum = _sc_gather_reduce(node_emb_table, ids, neigh_flat)

    tb = jnp.broadcast_to(node_interact_times[:, None], (B, HID))
    wsp1 = W_sp[:HID]
    wsp2 = W_sp[HID:] * (1.0 / NNB)
    wf1 = W_f[:HID]
    wf2 = W_f[HID:]
    w1a = W1[:HID]
    w1b = W1[HID:2 * HID]
    w1c = W1[2 * HID:]

    return _tc_dense(
        self_emb, nsum, tb,
        wsp1, wsp2, b_sp.reshape(1, HID),
        w_t.reshape(1, HID), b_t.reshape(1, HID),
        wf1, wf2, bf := b_f.reshape(1, HID),
        w1a, w1b, w1c, b1.reshape(1, HID),
        W2, b2.reshape(1, HID))


# speedup vs baseline: 2.6464x; 2.6464x over previous
import functools

import jax
import jax.numpy as jnp
from jax import lax
from jax.experimental import pallas as pl
from jax.experimental.pallas import tpu as pltpu
from jax.experimental.pallas import tpu_sc as plsc

B = 4096
HID = 128
NNB = 20

_NC = 2
_NS = 16
_NW = _NC * _NS
_BW = B // _NW
_CH = 16
_NCHUNK = _BW // _CH
_LANES = 8


def _sc_gather_reduce(table, node_ids, neigh_flat):
    mesh = plsc.VectorSubcoreMesh(core_axis_name="c", subcore_axis_name="s")

    @functools.partial(
        pl.kernel,
        mesh=mesh,
        out_type=[
            jax.ShapeDtypeStruct((B, HID), jnp.float32),
            jax.ShapeDtypeStruct((B, HID), jnp.float32),
        ],
        scratch_types=[
            pltpu.VMEM((_BW,), jnp.int32),
            pltpu.VMEM((_BW * NNB,), jnp.int32),
            pltpu.VMEM((_BW, HID), jnp.float32),
            pltpu.VMEM((_CH * NNB, HID), jnp.float32),
            pltpu.VMEM((_CH * NNB, HID), jnp.float32),
            pltpu.VMEM((_BW, HID), jnp.float32),
            pltpu.SemaphoreType.DMA,
            pltpu.SemaphoreType.DMA,
            pltpu.SemaphoreType.DMA,
        ],
    )
    def k(table_hbm, ids_hbm, nbr_hbm, self_out, sum_out,
          idx_v, nidx_v, self_v, bufa, bufb, sum_v, sem_self, sema, semb):
        wid = lax.axis_index("s") * _NC + lax.axis_index("c")
        base = wid * _BW

        pltpu.sync_copy(ids_hbm.at[pl.ds(base, _BW)], idx_v)
        pltpu.sync_copy(nbr_hbm.at[pl.ds(base * NNB, _BW * NNB)], nidx_v)

        self_cp = pltpu.async_copy(table_hbm.at[idx_v], self_v, sem_self)

        bufs = (bufa, bufb)
        sems = (sema, semb)
        pend = [None, None]
        pend[0] = pltpu.async_copy(
            table_hbm.at[nidx_v.at[pl.ds(0, _CH * NNB)]], bufs[0], sems[0])

        for c in range(_NCHUNK):
            cur = c % 2
            if c + 1 < _NCHUNK:
                nxt = (c + 1) % 2
                pend[nxt] = pltpu.async_copy(
                    table_hbm.at[nidx_v.at[pl.ds((c + 1) * _CH * NNB, _CH * NNB)]],
                    bufs[nxt], sems[nxt])
            pend[cur].wait()
            buf = bufs[cur]

            def body(i, carry, buf=buf, c=c):
                accs = [jnp.zeros((16,), jnp.float32) for _ in range(_LANES)]
                for j in range(NNB):
                    row = i * NNB + j
                    for g in range(_LANES):
                        accs[g] = accs[g] + buf[row, pl.ds(g * 16, 16)]
                for g in range(_LANES):
                    sum_v[c * _CH + i, pl.ds(g * 16, 16)] = accs[g]
                return carry

            lax.fori_loop(0, _CH, body, 0)

        self_cp.wait()
        pltpu.sync_copy(self_v, self_out.at[pl.ds(base, _BW)])
        pltpu.sync_copy(sum_v, sum_out.at[pl.ds(base, _BW)])

    return k(table, node_ids, neigh_flat)


_TBLK = 512
_TGRID = B // _TBLK


def _tc_body(self_ref, nsum_ref, tb_ref,
             wsp1_ref, wsp2_ref, bsp_ref, wt_ref, bt_ref,
             wf1_ref, wf2_ref, bf_ref,
             w1a_ref, w1b_ref, w1c_ref, b1_ref, w2_ref, b2_ref, out_ref):
    spatial = jnp.maximum(
        jnp.dot(self_ref[...], wsp1_ref[...], preferred_element_type=jnp.float32)
        + jnp.dot(nsum_ref[...], wsp2_ref[...], preferred_element_type=jnp.float32)
        + bsp_ref[...], 0.0)
    temporal = jnp.cos(tb_ref[...] * wt_ref[...] + bt_ref[...])
    fused = jnp.maximum(
        jnp.dot(spatial, wf1_ref[...], preferred_element_type=jnp.float32)
        + jnp.dot(temporal, wf2_ref[...], preferred_element_type=jnp.float32)
        + bf_ref[...], 0.0)
    h = jnp.maximum(
        jnp.dot(spatial, w1a_ref[...], preferred_element_type=jnp.float32)
        + jnp.dot(temporal, w1b_ref[...], preferred_element_type=jnp.float32)
        + jnp.dot(fused, w1c_ref[...], preferred_element_type=jnp.float32)
        + b1_ref[...], 0.0)
    out_ref[...] = (
        jnp.dot(h, w2_ref[...], preferred_element_type=jnp.float32)
        + b2_ref[...])


def _tc_dense(self_emb, nsum, tb, wsp1, wsp2, bsp, wt, bt,
              wf1, wf2, bf, w1a, w1b, w1c, b1, w2, b2):
    row_spec = pl.BlockSpec((_TBLK, HID), lambda i: (i, 0))

    def wspec(shape):
        return pl.BlockSpec(shape, lambda i, _s=len(shape): (0,) * _s)

    return pl.pallas_call(
        _tc_body,
        grid=(_TGRID,),
        in_specs=[
            row_spec, row_spec, row_spec,
            wspec(wsp1.shape), wspec(wsp2.shape), wspec(bsp.shape),
            wspec(wt.shape), wspec(bt.shape),
            wspec(wf1.shape), wspec(wf2.shape), wspec(bf.shape),
            wspec(w1a.shape), wspec(w1b.shape), wspec(w1c.shape),
            wspec(b1.shape), wspec(w2.shape), wspec(b2.shape),
        ],
        out_specs=row_spec,
        out_shape=jax.ShapeDtypeStruct((B, HID), jnp.float32),
    )(self_emb, nsum, tb, wsp1, wsp2, bsp, wt, bt,
      wf1, wf2, bf, w1a, w1b, w1c, b1, w2, b2)


def kernel(node_ids, node_interact_times, raw_node_features, node_emb_table,
           neighbor_ids, W_sp, b_sp, w_t, b_t, W_f, b_f, W1, b1, W2, b2):
    ids = node_ids.astype(jnp.int32)
    neigh_flat = neighbor_ids.astype(jnp.int32).reshape(B * NNB)

    self_emb, nsum = _sc_gather_reduce(node_emb_table, ids, neigh_flat)

    tb = jnp.broadcast_to(node_interact_times[:, None], (B, HID))
    wsp1 = W_sp[:HID]
    wsp2 = W_sp[HID:] * (1.0 / NNB)
    wf1 = W_f[:HID]
    wf2 = W_f[HID:]
    w1a = W1[:HID]
    w1b = W1[HID:2 * HID]
    w1c = W1[2 * HID:]

    return _tc_dense(
        self_emb, nsum, tb,
        wsp1, wsp2, b_sp.reshape(1, HID),
        w_t.reshape(1, HID), b_t.reshape(1, HID),
        wf1, wf2, b_f.reshape(1, HID),
        w1a, w1b, w1c, b1.reshape(1, HID),
        W2, b2.reshape(1, HID))


# speedup vs baseline: 2.8720x; 1.0853x over previous
import functools

import jax
import jax.numpy as jnp
from jax import lax
from jax.experimental import pallas as pl
from jax.experimental.pallas import tpu as pltpu
from jax.experimental.pallas import tpu_sc as plsc

B = 4096
HID = 128
NNB = 20

_NC = 2
_NS = 16
_NW = _NC * _NS
_BW = B // _NW
_CH = 16
_NCHUNK = _BW // _CH
_LANES = 8


def _sc_gather_reduce(table, node_ids, neigh_flat):
    mesh = plsc.VectorSubcoreMesh(core_axis_name="c", subcore_axis_name="s")

    @functools.partial(
        pl.kernel,
        mesh=mesh,
        out_type=[
            jax.ShapeDtypeStruct((B, HID), jnp.float32),
            jax.ShapeDtypeStruct((B, HID), jnp.float32),
        ],
        scratch_types=[
            pltpu.VMEM((_BW,), jnp.int32),
            pltpu.VMEM((_BW * NNB,), jnp.int32),
            pltpu.VMEM((_BW, HID), jnp.float32),
            pltpu.VMEM((_CH * NNB, HID), jnp.float32),
            pltpu.VMEM((_CH * NNB, HID), jnp.float32),
            pltpu.VMEM((_BW, HID), jnp.float32),
            pltpu.SemaphoreType.DMA,
            pltpu.SemaphoreType.DMA,
            pltpu.SemaphoreType.DMA,
        ],
    )
    def k(table_hbm, ids_hbm, nbr_hbm, self_out, sum_out,
          idx_v, nidx_v, self_v, bufa, bufb, sum_v, sem_self, sema, semb):
        wid = lax.axis_index("s") * _NC + lax.axis_index("c")
        base = wid * _BW

        pltpu.sync_copy(ids_hbm.at[pl.ds(base, _BW)], idx_v)
        pltpu.sync_copy(nbr_hbm.at[pl.ds(base * NNB, _BW * NNB)], nidx_v)

        self_cp = pltpu.async_copy(table_hbm.at[idx_v], self_v, sem_self)

        bufs = (bufa, bufb)
        sems = (sema, semb)
        pend = [None, None]
        pend[0] = pltpu.async_copy(
            table_hbm.at[nidx_v.at[pl.ds(0, _CH * NNB)]], bufs[0], sems[0])

        for c in range(_NCHUNK):
            cur = c % 2
            if c + 1 < _NCHUNK:
                nxt = (c + 1) % 2
                pend[nxt] = pltpu.async_copy(
                    table_hbm.at[nidx_v.at[pl.ds((c + 1) * _CH * NNB, _CH * NNB)]],
                    bufs[nxt], sems[nxt])
            pend[cur].wait()
            buf = bufs[cur]

            def body(i, carry, buf=buf, c=c):
                accs = [jnp.zeros((16,), jnp.float32) for _ in range(_LANES)]
                for j in range(NNB):
                    row = i * NNB + j
                    for g in range(_LANES):
                        accs[g] = accs[g] + buf[row, pl.ds(g * 16, 16)]
                for g in range(_LANES):
                    sum_v[c * _CH + i, pl.ds(g * 16, 16)] = accs[g]
                return carry

            lax.fori_loop(0, _CH, body, 0)

        self_cp.wait()
        pltpu.sync_copy(self_v, self_out.at[pl.ds(base, _BW)])
        pltpu.sync_copy(sum_v, sum_out.at[pl.ds(base, _BW)])

    return k(table, node_ids, neigh_flat)


_TBLK = 512
_TGRID = B // _TBLK


def _temporal_body(tb_ref, wt_ref, bt_ref, wtb_ref, out_ref):
    temporal = jnp.cos(tb_ref[...] * wt_ref[...] + bt_ref[...])
    out_ref[...] = jnp.dot(temporal, wtb_ref[...],
                           preferred_element_type=jnp.float32)


def _tc_temporal(tb, wt, bt, wtb):
    def wspec(shape):
        return pl.BlockSpec(shape, lambda i, _s=len(shape): (0,) * _s)

    return pl.pallas_call(
        _temporal_body,
        grid=(_TGRID,),
        in_specs=[
            pl.BlockSpec((_TBLK, HID), lambda i: (i, 0)),
            wspec(wt.shape), wspec(bt.shape), wspec(wtb.shape),
        ],
        out_specs=pl.BlockSpec((_TBLK, 2 * HID), lambda i: (i, 0)),
        out_shape=jax.ShapeDtypeStruct((B, 2 * HID), jnp.float32),
    )(tb, wt, bt, wtb)


def _tc_body(self_ref, nsum_ref, tboth_ref,
             wsp1_ref, wsp2_ref, bsp_ref,
             wf1_ref, bf_ref,
             w1a_ref, w1c_ref, b1_ref, w2_ref, b2_ref, out_ref):
    spatial = jnp.maximum(
        jnp.dot(self_ref[...], wsp1_ref[...], preferred_element_type=jnp.float32)
        + jnp.dot(nsum_ref[...], wsp2_ref[...], preferred_element_type=jnp.float32)
        + bsp_ref[...], 0.0)
    tboth = tboth_ref[...]
    fused = jnp.maximum(
        jnp.dot(spatial, wf1_ref[...], preferred_element_type=jnp.float32)
        + tboth[:, :HID] + bf_ref[...], 0.0)
    h = jnp.maximum(
        jnp.dot(spatial, w1a_ref[...], preferred_element_type=jnp.float32)
        + jnp.dot(fused, w1c_ref[...], preferred_element_type=jnp.float32)
        + tboth[:, HID:] + b1_ref[...], 0.0)
    out_ref[...] = (
        jnp.dot(h, w2_ref[...], preferred_element_type=jnp.float32)
        + b2_ref[...])


def _tc_dense(self_emb, nsum, tboth, wsp1, wsp2, bsp,
              wf1, bf, w1a, w1c, b1, w2, b2):
    row_spec = pl.BlockSpec((_TBLK, HID), lambda i: (i, 0))

    def wspec(shape):
        return pl.BlockSpec(shape, lambda i, _s=len(shape): (0,) * _s)

    return pl.pallas_call(
        _tc_body,
        grid=(_TGRID,),
        in_specs=[
            row_spec, row_spec,
            pl.BlockSpec((_TBLK, 2 * HID), lambda i: (i, 0)),
            wspec(wsp1.shape), wspec(wsp2.shape), wspec(bsp.shape),
            wspec(wf1.shape), wspec(bf.shape),
            wspec(w1a.shape), wspec(w1c.shape),
            wspec(b1.shape), wspec(w2.shape), wspec(b2.shape),
        ],
        out_specs=row_spec,
        out_shape=jax.ShapeDtypeStruct((B, HID), jnp.float32),
    )(self_emb, nsum, tboth, wsp1, wsp2, bsp,
      wf1, bf, w1a, w1c, b1, w2, b2)


def kernel(node_ids, node_interact_times, raw_node_features, node_emb_table,
           neighbor_ids, W_sp, b_sp, w_t, b_t, W_f, b_f, W1, b1, W2, b2):
    ids = node_ids.astype(jnp.int32)
    neigh_flat = neighbor_ids.astype(jnp.int32).reshape(B * NNB)

    self_emb, nsum = _sc_gather_reduce(node_emb_table, ids, neigh_flat)

    tb = jnp.broadcast_to(node_interact_times[:, None], (B, HID))
    wsp1 = W_sp[:HID]
    wsp2 = W_sp[HID:] * (1.0 / NNB)
    wf1 = W_f[:HID]
    wf2 = W_f[HID:]
    w1a = W1[:HID]
    w1b = W1[HID:2 * HID]
    w1c = W1[2 * HID:]
    wtb = jnp.concatenate([wf2, w1b], axis=1)

    tboth = _tc_temporal(tb, w_t.reshape(1, HID), b_t.reshape(1, HID), wtb)

    return _tc_dense(
        self_emb, nsum, tboth,
        wsp1, wsp2, b_sp.reshape(1, HID),
        wf1, b_f.reshape(1, HID),
        w1a, w1c, b1.reshape(1, HID),
        W2, b2.reshape(1, HID))


# speedup vs baseline: 2.9335x; 1.0214x over previous
import functools

import jax
import jax.numpy as jnp
import numpy as np
from jax import lax
from jax.experimental import pallas as pl
from jax.experimental.pallas import tpu as pltpu
from jax.experimental.pallas import tpu_sc as plsc

B = 4096
HID = 128
NNB = 20

_NC = 2
_NS = 16
_NW = _NC * _NS
_BW = B // _NW
_CH = 16
_NCHUNK = _BW // _CH
_LANES = 8
_HWORDS = HID // 2

_PERM_LO = np.array([32 * (c // 16) + (c % 16) for c in range(_HWORDS)],
                    dtype=np.int32)
_PERM_HI = _PERM_LO + 16


def _bf16_pack_words(a, b):
    ba = lax.bitcast_convert_type(a, jnp.int32)
    bb = lax.bitcast_convert_type(b, jnp.int32)
    ra = ba + 0x7FFF + (lax.shift_right_logical(ba, 16) & 1)
    rb = bb + 0x7FFF + (lax.shift_right_logical(bb, 16) & 1)
    lo = lax.shift_right_logical(ra, 16)
    hi = rb & jnp.int32(-65536)
    return lo | hi


def _sc_gather_reduce(table, node_ids, neigh_flat):
    mesh = plsc.VectorSubcoreMesh(core_axis_name="c", subcore_axis_name="s")

    @functools.partial(
        pl.kernel,
        mesh=mesh,
        out_type=jax.ShapeDtypeStruct((B, HID), jnp.int32),
        scratch_types=[
            pltpu.VMEM((_BW,), jnp.int32),
            pltpu.VMEM((_BW * NNB,), jnp.int32),
            pltpu.VMEM((_BW, HID), jnp.float32),
            pltpu.VMEM((_CH * NNB, HID), jnp.float32),
            pltpu.VMEM((_CH * NNB, HID), jnp.float32),
            pltpu.VMEM((_BW, HID), jnp.int32),
            pltpu.SemaphoreType.DMA,
            pltpu.SemaphoreType.DMA,
            pltpu.SemaphoreType.DMA,
        ],
    )
    def k(table_hbm, ids_hbm, nbr_hbm, packed_out,
          idx_v, nidx_v, self_v, bufa, bufb, packed_v,
          sem_self, sema, semb):
        wid = lax.axis_index("s") * _NC + lax.axis_index("c")
        base = wid * _BW

        pltpu.sync_copy(ids_hbm.at[pl.ds(base, _BW)], idx_v)
        pltpu.sync_copy(nbr_hbm.at[pl.ds(base * NNB, _BW * NNB)], nidx_v)

        self_cp = pltpu.async_copy(table_hbm.at[idx_v], self_v, sem_self)

        def reduce_chunk(c, buf):
            def body(i, carry):
                accs = [jnp.zeros((16,), jnp.float32) for _ in range(_LANES)]
                for j in range(NNB):
                    row = i * NNB + j
                    for g in range(_LANES):
                        accs[g] = accs[g] + buf[row, pl.ds(g * 16, 16)]
                for q in range(_LANES // 2):
                    w = _bf16_pack_words(accs[2 * q], accs[2 * q + 1])
                    packed_v[c * _CH + i, pl.ds(_HWORDS + 16 * q, 16)] = w
                return carry

            lax.fori_loop(0, _CH, body, 0)

        def gather_chunk(c, buf, sem):
            return pltpu.async_copy(
                table_hbm.at[nidx_v.at[pl.ds(c * _CH * NNB, _CH * NNB)]],
                buf, sem)

        bufs = (bufa, bufb)
        sems = (sema, semb)
        pend = [None, None]
        pend[0] = gather_chunk(0, bufs[0], sems[0])
        for c in range(_NCHUNK):
            cur = c % 2
            if c + 1 < _NCHUNK:
                nxt = (c + 1) % 2
                pend[nxt] = gather_chunk(c + 1, bufs[nxt], sems[nxt])
            pend[cur].wait()
            reduce_chunk(c, bufs[cur])

        self_cp.wait()

        def pack_self(i, carry):
            for q in range(_LANES // 2):
                a = self_v[i, pl.ds(32 * q, 16)]
                b = self_v[i, pl.ds(32 * q + 16, 16)]
                packed_v[i, pl.ds(16 * q, 16)] = _bf16_pack_words(a, b)
            return carry

        lax.fori_loop(0, _BW, pack_self, 0)

        pltpu.sync_copy(packed_v, packed_out.at[pl.ds(base, _BW)])

    return k(table, node_ids, neigh_flat)


_TBLK = 1024
_TGRID = B // _TBLK


def _temporal_body(tb_ref, wt_ref, bt_ref, wtb_ref, out_ref):
    temporal = jnp.cos(tb_ref[...] * wt_ref[...] + bt_ref[...])
    out_ref[...] = jnp.dot(temporal, wtb_ref[...],
                           preferred_element_type=jnp.float32
                           ).astype(jnp.bfloat16)


def _tc_temporal(tb, wt, bt, wtb):
    def wspec(shape):
        return pl.BlockSpec(shape, lambda i, _s=len(shape): (0,) * _s)

    return pl.pallas_call(
        _temporal_body,
        grid=(_TGRID,),
        in_specs=[
            pl.BlockSpec((_TBLK, HID), lambda i: (i, 0)),
            wspec(wt.shape), wspec(bt.shape), wspec(wtb.shape),
        ],
        out_specs=pl.BlockSpec((_TBLK, 2 * HID), lambda i: (i, 0)),
        out_shape=jax.ShapeDtypeStruct((B, 2 * HID), jnp.bfloat16),
    )(tb, wt, bt, wtb)


def _unpack_halves(vi32):
    lo = lax.bitcast_convert_type(lax.shift_left(vi32, 16), jnp.float32)
    hi = lax.bitcast_convert_type(vi32 & jnp.int32(-65536), jnp.float32)
    return lo, hi


def _tc_body(packed_ref, tboth_ref,
             wsp1a_ref, wsp1b_ref, wsp2a_ref, wsp2b_ref, bsp_ref,
             wf1_ref, bf_ref,
             w1a_ref, w1c_ref, b1_ref, w2_ref, b2_ref, out_ref):
    packed = packed_ref[...]
    sa, sb = _unpack_halves(packed[:, :_HWORDS])
    na, nb = _unpack_halves(packed[:, _HWORDS:])
    spatial = jnp.maximum(
        jnp.dot(sa, wsp1a_ref[...], preferred_element_type=jnp.float32)
        + jnp.dot(sb, wsp1b_ref[...], preferred_element_type=jnp.float32)
        + jnp.dot(na, wsp2a_ref[...], preferred_element_type=jnp.float32)
        + jnp.dot(nb, wsp2b_ref[...], preferred_element_type=jnp.float32)
        + bsp_ref[...], 0.0)
    tboth = tboth_ref[...].astype(jnp.float32)
    fused = jnp.maximum(
        jnp.dot(spatial, wf1_ref[...], preferred_element_type=jnp.float32)
        + tboth[:, :HID] + bf_ref[...], 0.0)
    h = jnp.maximum(
        jnp.dot(spatial, w1a_ref[...], preferred_element_type=jnp.float32)
        + jnp.dot(fused, w1c_ref[...], preferred_element_type=jnp.float32)
        + tboth[:, HID:] + b1_ref[...], 0.0)
    out_ref[...] = (
        jnp.dot(h, w2_ref[...], preferred_element_type=jnp.float32)
        + b2_ref[...])


def _tc_dense(packed, tboth, wsp1a, wsp1b, wsp2a, wsp2b, bsp,
              wf1, bf, w1a, w1c, b1, w2, b2):
    packed_spec = pl.BlockSpec((_TBLK, HID), lambda i: (i, 0))

    def wspec(shape):
        return pl.BlockSpec(shape, lambda i, _s=len(shape): (0,) * _s)

    return pl.pallas_call(
        _tc_body,
        grid=(_TGRID,),
        in_specs=[
            packed_spec,
            pl.BlockSpec((_TBLK, 2 * HID), lambda i: (i, 0)),
            wspec(wsp1a.shape), wspec(wsp1b.shape),
            wspec(wsp2a.shape), wspec(wsp2b.shape), wspec(bsp.shape),
            wspec(wf1.shape), wspec(bf.shape),
            wspec(w1a.shape), wspec(w1c.shape),
            wspec(b1.shape), wspec(w2.shape), wspec(b2.shape),
        ],
        out_specs=pl.BlockSpec((_TBLK, HID), lambda i: (i, 0)),
        out_shape=jax.ShapeDtypeStruct((B, HID), jnp.float32),
    )(packed, tboth, wsp1a, wsp1b, wsp2a, wsp2b, bsp,
      wf1, bf, w1a, w1c, b1, w2, b2)


def kernel(node_ids, node_interact_times, raw_node_features, node_emb_table,
           neighbor_ids, W_sp, b_sp, w_t, b_t, W_f, b_f, W1, b1, W2, b2):
    ids = node_ids.astype(jnp.int32)
    neigh_flat = neighbor_ids.astype(jnp.int32).reshape(B * NNB)

    packed = _sc_gather_reduce(node_emb_table, ids, neigh_flat)

    tb = jnp.broadcast_to(node_interact_times[:, None], (B, HID))
    plo = jnp.asarray(_PERM_LO)
    phi = jnp.asarray(_PERM_HI)
    wsp1 = W_sp[:HID]
    wsp2 = W_sp[HID:] * (1.0 / NNB)
    wf1 = W_f[:HID]
    wf2 = W_f[HID:]
    w1a = W1[:HID]
    w1b = W1[HID:2 * HID]
    w1c = W1[2 * HID:]
    wtb = jnp.concatenate([wf2, w1b], axis=1)

    tboth = _tc_temporal(tb, w_t.reshape(1, HID), b_t.reshape(1, HID), wtb)

    return _tc_dense(
        packed, tboth,
        wsp1[plo], wsp1[phi], wsp2[plo], wsp2[phi], b_sp.reshape(1, HID),
        wf1, b_f.reshape(1, HID),
        w1a, w1c, b1.reshape(1, HID),
        W2, b2.reshape(1, HID))


# speedup vs baseline: 2.9809x; 1.0162x over previous
import functools

import jax
import jax.numpy as jnp
from jax import lax
from jax.experimental import pallas as pl
from jax.experimental.pallas import tpu as pltpu
from jax.experimental.pallas import tpu_sc as plsc

B = 4096
HID = 128
NNB = 20

_NC = 2
_NS = 16
_NW = _NC * _NS
_BW = B // _NW
_CH = 16
_NCHUNK = _BW // _CH
_LANES = 8


def _sc_gather_reduce(table, node_ids, neigh_flat):
    mesh = plsc.VectorSubcoreMesh(core_axis_name="c", subcore_axis_name="s")

    @functools.partial(
        pl.kernel,
        mesh=mesh,
        out_type=[
            jax.ShapeDtypeStruct((B, HID), jnp.float32),
            jax.ShapeDtypeStruct((B, HID), jnp.float32),
        ],
        scratch_types=[
            pltpu.VMEM((_BW,), jnp.int32),
            pltpu.VMEM((_BW * NNB,), jnp.int32),
            pltpu.VMEM((_BW, HID), jnp.float32),
            pltpu.VMEM((_CH * NNB, HID), jnp.float32),
            pltpu.VMEM((_CH * NNB, HID), jnp.float32),
            pltpu.VMEM((_BW, HID), jnp.float32),
            pltpu.SemaphoreType.DMA,
            pltpu.SemaphoreType.DMA,
            pltpu.SemaphoreType.DMA,
            pltpu.SemaphoreType.DMA,
        ],
    )
    def k(table_hbm, ids_hbm, nbr_hbm, self_out, sum_out,
          idx_v, nidx_v, self_v, bufa, bufb, sum_v,
          sem_self, sema, semb, sem_out):
        wid = lax.axis_index("s") * _NC + lax.axis_index("c")
        base = wid * _BW

        pltpu.sync_copy(ids_hbm.at[pl.ds(base, _BW)], idx_v)
        pltpu.sync_copy(nbr_hbm.at[pl.ds(base * NNB, _BW * NNB)], nidx_v)

        self_cp = pltpu.async_copy(table_hbm.at[idx_v], self_v, sem_self)

        def reduce_chunk(c, buf):
            def body(i, carry):
                accs = [jnp.zeros((16,), jnp.float32) for _ in range(_LANES)]
                for j in range(NNB):
                    row = i * NNB + j
                    for g in range(_LANES):
                        accs[g] = accs[g] + buf[row, pl.ds(g * 16, 16)]
                for g in range(_LANES):
                    sum_v[c * _CH + i, pl.ds(g * 16, 16)] = accs[g]
                return carry

            lax.fori_loop(0, _CH, body, 0)

        def gather_chunk(c, buf, sem):
            return pltpu.async_copy(
                table_hbm.at[nidx_v.at[pl.ds(c * _CH * NNB, _CH * NNB)]],
                buf, sem)

        bufs = (bufa, bufb)
        sems = (sema, semb)
        pend = [None, None]
        out_cps = []
        pend[0] = gather_chunk(0, bufs[0], sems[0])
        for c in range(_NCHUNK):
            cur = c % 2
            if c + 1 < _NCHUNK:
                nxt = (c + 1) % 2
                pend[nxt] = gather_chunk(c + 1, bufs[nxt], sems[nxt])
            pend[cur].wait()
            reduce_chunk(c, bufs[cur])
            out_cps.append(pltpu.async_copy(
                sum_v.at[pl.ds(c * _CH, _CH)],
                sum_out.at[pl.ds(base + c * _CH, _CH)], sem_out))
            if c == 0:
                self_cp.wait()
                out_cps.append(pltpu.async_copy(
                    self_v, self_out.at[pl.ds(base, _BW)], sem_out))

        for cp in out_cps:
            cp.wait()

    return k(table, node_ids, neigh_flat)


_TBLK = 1024
_TGRID = B // _TBLK


def _temporal_body(tb_ref, wt_ref, bt_ref, wtb_ref, out_ref):
    temporal = jnp.cos(tb_ref[...] * wt_ref[...] + bt_ref[...])
    out_ref[...] = jnp.dot(temporal, wtb_ref[...],
                           preferred_element_type=jnp.float32
                           ).astype(jnp.bfloat16)


def _tc_temporal(tb, wt, bt, wtb):
    def wspec(shape):
        return pl.BlockSpec(shape, lambda i, _s=len(shape): (0,) * _s)

    return pl.pallas_call(
        _temporal_body,
        grid=(_TGRID,),
        in_specs=[
            pl.BlockSpec((_TBLK, HID), lambda i: (i, 0)),
            wspec(wt.shape), wspec(bt.shape), wspec(wtb.shape),
        ],
        out_specs=pl.BlockSpec((_TBLK, 2 * HID), lambda i: (i, 0)),
        out_shape=jax.ShapeDtypeStruct((B, 2 * HID), jnp.bfloat16),
    )(tb, wt, bt, wtb)


def _tc_body(self_ref, nsum_ref, tboth_ref,
             wsp1_ref, wsp2_ref, bsp_ref,
             wf1_ref, bf_ref,
             w1a_ref, w1c_ref, b1_ref, w2_ref, b2_ref, out_ref):
    spatial = jnp.maximum(
        jnp.dot(self_ref[...], wsp1_ref[...], preferred_element_type=jnp.float32)
        + jnp.dot(nsum_ref[...], wsp2_ref[...], preferred_element_type=jnp.float32)
        + bsp_ref[...], 0.0)
    tboth = tboth_ref[...].astype(jnp.float32)
    fused = jnp.maximum(
        jnp.dot(spatial, wf1_ref[...], preferred_element_type=jnp.float32)
        + tboth[:, :HID] + bf_ref[...], 0.0)
    h = jnp.maximum(
        jnp.dot(spatial, w1a_ref[...], preferred_element_type=jnp.float32)
        + jnp.dot(fused, w1c_ref[...], preferred_element_type=jnp.float32)
        + tboth[:, HID:] + b1_ref[...], 0.0)
    out_ref[...] = (
        jnp.dot(h, w2_ref[...], preferred_element_type=jnp.float32)
        + b2_ref[...])


def _tc_dense(self_emb, nsum, tboth, wsp1, wsp2, bsp,
              wf1, bf, w1a, w1c, b1, w2, b2):
    row_spec = pl.BlockSpec((_TBLK, HID), lambda i: (i, 0))

    def wspec(shape):
        return pl.BlockSpec(shape, lambda i, _s=len(shape): (0,) * _s)

    return pl.pallas_call(
        _tc_body,
        grid=(_TGRID,),
        in_specs=[
            row_spec, row_spec,
            pl.BlockSpec((_TBLK, 2 * HID), lambda i: (i, 0)),
            wspec(wsp1.shape), wspec(wsp2.shape), wspec(bsp.shape),
            wspec(wf1.shape), wspec(bf.shape),
            wspec(w1a.shape), wspec(w1c.shape),
            wspec(b1.shape), wspec(w2.shape), wspec(b2.shape),
        ],
        out_specs=row_spec,
        out_shape=jax.ShapeDtypeStruct((B, HID), jnp.float32),
    )(self_emb, nsum, tboth, wsp1, wsp2, bsp,
      wf1, bf, w1a, w1c, b1, w2, b2)


def kernel(node_ids, node_interact_times, raw_node_features, node_emb_table,
           neighbor_ids, W_sp, b_sp, w_t, b_t, W_f, b_f, W1, b1, W2, b2):
    ids = node_ids.astype(jnp.int32)
    neigh_flat = neighbor_ids.astype(jnp.int32).reshape(B * NNB)

    self_emb, nsum = _sc_gather_reduce(node_emb_table, ids, neigh_flat)

    tb = jnp.broadcast_to(node_interact_times[:, None], (B, HID))
    wsp1 = W_sp[:HID]
    wsp2 = W_sp[HID:] * (1.0 / NNB)
    wf1 = W_f[:HID]
    wf2 = W_f[HID:]
    w1a = W1[:HID]
    w1b = W1[HID:2 * HID]
    w1c = W1[2 * HID:]
    wtb = jnp.concatenate([wf2, w1b], axis=1)

    tboth = _tc_temporal(tb, w_t.reshape(1, HID), b_t.reshape(1, HID), wtb)

    return _tc_dense(
        self_emb, nsum, tboth,
        wsp1, wsp2, b_sp.reshape(1, HID),
        wf1, b_f.reshape(1, HID),
        w1a, w1c, b1.reshape(1, HID),
        W2, b2.reshape(1, HID))


# speedup vs baseline: 3.0033x; 1.0075x over previous
import functools

import jax
import jax.numpy as jnp
from jax import lax
from jax.experimental import pallas as pl
from jax.experimental.pallas import tpu as pltpu
from jax.experimental.pallas import tpu_sc as plsc

B = 4096
HID = 128
NNB = 20

_NC = 2
_NS = 16
_NW = _NC * _NS
_BW = B // _NW
_CH = 16
_NCHUNK = _BW // _CH
_LANES = 8


def _sc_gather_reduce(table, node_ids, neigh_flat):
    mesh = plsc.VectorSubcoreMesh(core_axis_name="c", subcore_axis_name="s")

    @functools.partial(
        pl.kernel,
        mesh=mesh,
        out_type=[
            jax.ShapeDtypeStruct((B, HID), jnp.float32),
            jax.ShapeDtypeStruct((B, HID), jnp.float32),
        ],
        scratch_types=[
            pltpu.VMEM((_BW,), jnp.int32),
            pltpu.VMEM((_BW * NNB,), jnp.int32),
            pltpu.VMEM((_BW, HID), jnp.float32),
            pltpu.VMEM((_CH * NNB, HID), jnp.float32),
            pltpu.VMEM((_CH * NNB, HID), jnp.float32),
            pltpu.VMEM((_BW, HID), jnp.float32),
            pltpu.SemaphoreType.DMA,
            pltpu.SemaphoreType.DMA,
            pltpu.SemaphoreType.DMA,
            pltpu.SemaphoreType.DMA,
        ],
    )
    def k(table_hbm, ids_hbm, nbr_hbm, self_out, sum_out,
          idx_v, nidx_v, self_v, bufa, bufb, sum_v,
          sem_self, sema, semb, sem_out):
        wid = lax.axis_index("s") * _NC + lax.axis_index("c")
        base = wid * _BW

        pltpu.sync_copy(ids_hbm.at[pl.ds(base, _BW)], idx_v)
        pltpu.sync_copy(nbr_hbm.at[pl.ds(base * NNB, _BW * NNB)], nidx_v)

        self_cp = pltpu.async_copy(table_hbm.at[idx_v], self_v, sem_self)

        def reduce_chunk(c, buf):
            def body(i, carry):
                accs = [jnp.zeros((16,), jnp.float32) for _ in range(_LANES)]
                for j in range(NNB):
                    row = i * NNB + j
                    for g in range(_LANES):
                        accs[g] = accs[g] + buf[row, pl.ds(g * 16, 16)]
                for g in range(_LANES):
                    sum_v[c * _CH + i, pl.ds(g * 16, 16)] = accs[g]
                return carry

            lax.fori_loop(0, _CH, body, 0)

        def gather_chunk(c, buf, sem):
            return pltpu.async_copy(
                table_hbm.at[nidx_v.at[pl.ds(c * _CH * NNB, _CH * NNB)]],
                buf, sem)

        bufs = (bufa, bufb)
        sems = (sema, semb)
        pend = [None, None]
        self_wb = None
        pend[0] = gather_chunk(0, bufs[0], sems[0])
        for c in range(_NCHUNK):
            cur = c % 2
            if c + 1 < _NCHUNK:
                nxt = (c + 1) % 2
                pend[nxt] = gather_chunk(c + 1, bufs[nxt], sems[nxt])
            pend[cur].wait()
            reduce_chunk(c, bufs[cur])
            if c == 0:
                self_cp.wait()
                self_wb = pltpu.async_copy(
                    self_v, self_out.at[pl.ds(base, _BW)], sem_out)

        pltpu.sync_copy(sum_v, sum_out.at[pl.ds(base, _BW)])
        self_wb.wait()

    return k(table, node_ids, neigh_flat)


_TBLK = 1024
_TGRID = B // _TBLK


def _temporal_body(tb_ref, wt_ref, bt_ref, wtb_ref, out_ref):
    temporal = jnp.cos(tb_ref[...] * wt_ref[...] + bt_ref[...])
    out_ref[...] = jnp.dot(temporal, wtb_ref[...],
                           preferred_element_type=jnp.float32
                           ).astype(jnp.bfloat16)


def _tc_temporal(tb, wt, bt, wtb):
    def wspec(shape):
        return pl.BlockSpec(shape, lambda i, _s=len(shape): (0,) * _s)

    return pl.pallas_call(
        _temporal_body,
        grid=(_TGRID,),
        in_specs=[
            pl.BlockSpec((_TBLK, HID), lambda i: (i, 0)),
            wspec(wt.shape), wspec(bt.shape), wspec(wtb.shape),
        ],
        out_specs=pl.BlockSpec((_TBLK, 2 * HID), lambda i: (i, 0)),
        out_shape=jax.ShapeDtypeStruct((B, 2 * HID), jnp.bfloat16),
    )(tb, wt, bt, wtb)


def _tc_body(self_ref, nsum_ref, tboth_ref,
             wsp1_ref, wsp2_ref, bsp_ref,
             wf1_ref, bf_ref,
             w1a_ref, w1c_ref, b1_ref, w2_ref, b2_ref, out_ref):
    spatial = jnp.maximum(
        jnp.dot(self_ref[...], wsp1_ref[...], preferred_element_type=jnp.float32)
        + jnp.dot(nsum_ref[...], wsp2_ref[...], preferred_element_type=jnp.float32)
        + bsp_ref[...], 0.0)
    tboth = tboth_ref[...].astype(jnp.float32)
    fused = jnp.maximum(
        jnp.dot(spatial, wf1_ref[...], preferred_element_type=jnp.float32)
        + tboth[:, :HID] + bf_ref[...], 0.0)
    h = jnp.maximum(
        jnp.dot(spatial, w1a_ref[...], preferred_element_type=jnp.float32)
        + jnp.dot(fused, w1c_ref[...], preferred_element_type=jnp.float32)
        + tboth[:, HID:] + b1_ref[...], 0.0)
    out_ref[...] = (
        jnp.dot(h, w2_ref[...], preferred_element_type=jnp.float32)
        + b2_ref[...])


def _tc_dense(self_emb, nsum, tboth, wsp1, wsp2, bsp,
              wf1, bf, w1a, w1c, b1, w2, b2):
    row_spec = pl.BlockSpec((_TBLK, HID), lambda i: (i, 0))

    def wspec(shape):
        return pl.BlockSpec(shape, lambda i, _s=len(shape): (0,) * _s)

    return pl.pallas_call(
        _tc_body,
        grid=(_TGRID,),
        in_specs=[
            row_spec, row_spec,
            pl.BlockSpec((_TBLK, 2 * HID), lambda i: (i, 0)),
            wspec(wsp1.shape), wspec(wsp2.shape), wspec(bsp.shape),
            wspec(wf1.shape), wspec(bf.shape),
            wspec(w1a.shape), wspec(w1c.shape),
            wspec(b1.shape), wspec(w2.shape), wspec(b2.shape),
        ],
        out_specs=row_spec,
        out_shape=jax.ShapeDtypeStruct((B, HID), jnp.float32),
    )(self_emb, nsum, tboth, wsp1, wsp2, bsp,
      wf1, bf, w1a, w1c, b1, w2, b2)


def kernel(node_ids, node_interact_times, raw_node_features, node_emb_table,
           neighbor_ids, W_sp, b_sp, w_t, b_t, W_f, b_f, W1, b1, W2, b2):
    ids = node_ids.astype(jnp.int32)
    neigh_flat = neighbor_ids.astype(jnp.int32).reshape(B * NNB)

    self_emb, nsum = _sc_gather_reduce(node_emb_table, ids, neigh_flat)

    tb = jnp.broadcast_to(node_interact_times[:, None], (B, HID))
    wsp1 = W_sp[:HID]
    wsp2 = W_sp[HID:] * (1.0 / NNB)
    wf1 = W_f[:HID]
    wf2 = W_f[HID:]
    w1a = W1[:HID]
    w1b = W1[HID:2 * HID]
    w1c = W1[2 * HID:]
    wtb = jnp.concatenate([wf2, w1b], axis=1)

    tboth = _tc_temporal(tb, w_t.reshape(1, HID), b_t.reshape(1, HID), wtb)

    return _tc_dense(
        self_emb, nsum, tboth,
        wsp1, wsp2, b_sp.reshape(1, HID),
        wf1, b_f.reshape(1, HID),
        w1a, w1c, b1.reshape(1, HID),
        W2, b2.reshape(1, HID))
